# baseline (speedup 1.0000x reference)
import jax
import jax.numpy as jnp
from jax import lax
from jax.experimental import pallas as pl
from jax.experimental.pallas import tpu as pltpu
from jax.experimental.pallas import tpu_sc as plsc

N = 12_800_000
G = 128
NC, NS, L = 2, 16, 16
NW = NC * NS
CHUNK = N // NW
BLK = 20_000
NBLK = CHUNK // BLK
HALF = NBLK // 2
VPB = BLK // L
UNROLL = 8
MPAD = 80
EB = 128
NEG = -3.0e38

_mesh = plsc.VectorSubcoreMesh(core_axis_name="c", subcore_axis_name="s")
_params = pltpu.CompilerParams(needs_layout_passes=False)


def _wid():
    return lax.axis_index("s") * NC + lax.axis_index("c")


def _p1_body(x_hbm, i_hbm, pm_hbm, ps_hbm, meta_hbm,
             xb0, xb1, ibuf, ebuf, eidx, mbuf, accm, accs,
             sx0, sx1, se):
    wid = _wid()
    base = wid * CHUNK
    lane = lax.iota(jnp.int32, L)
    lane0 = lane == 0
    xb, sx = [xb0, xb1], [sx0, sx1]

    for j in range(G // L):
        accm[pl.ds(j * L, L)] = jnp.full((L,), NEG, jnp.float32)
        accs[pl.ds(j * L, L)] = jnp.zeros((L,), jnp.float32)
    for j in range(MPAD // L):
        mbuf[pl.ds(j * L, L)] = jnp.full((L,), -1, jnp.int32)

    for k in range(EB // L):
        p = k * L + lane
        b = jnp.minimum(p >> 1, NBLK - 1)
        is_end = p & 1
        eidx[pl.ds(k * L, L)] = base + b * BLK + is_end * (BLK - 1)
    pltpu.async_copy(i_hbm.at[eidx], ebuf, se).wait()

    def start(j, off):
        pltpu.make_async_copy(x_hbm.at[pl.ds(off, BLK)], xb[j], sx[j]).start()

    def wait(j):
        pltpu.make_async_copy(x_hbm.at[pl.ds(0, BLK)], xb[j], sx[j]).wait()

    def merge(gvec, m_sc, s_sc):
        mold = plsc.load_gather(accm, [gvec])
        sold = plsc.load_gather(accs, [gvec])
        mnew = jnp.maximum(mold, m_sc)
        snew = sold * jnp.exp(mold - mnew) + s_sc * jnp.exp(m_sc - mnew)
        plsc.store_scatter(accm, [gvec], mnew, mask=lane0)
        plsc.store_scatter(accs, [gvec], snew, mask=lane0)

    def compute(j, b):
        xbuf = xb[j]
        g0 = ebuf[pl.ds(2 * b, L)][0]
        g1 = ebuf[pl.ds(2 * b, L)][1]
        bvec = jnp.full((L,), b, jnp.int32)

        def uniform():
            gvec = jnp.full((L,), g0, jnp.int32)
            macc = jnp.max(plsc.load_gather(accm, [gvec]))

            def sum_sweep(shift):
                @plsc.parallel_loop(
                    0, BLK, step=L, unroll=UNROLL,
                    carry=(jnp.full((L,), NEG, jnp.float32),
                           jnp.zeros((L,), jnp.float32)))
                def ms(o, carry):
                    m, s = carry
                    xv = xbuf[pl.ds(o, L)]
                    return jnp.maximum(m, xv), s + jnp.exp(xv - shift)
                mv, sv = ms
                return jnp.max(mv), jnp.sum(sv)

            def seeded():
                m_b, s = sum_sweep(macc)

                def ok():
                    merge(gvec, macc, s)

                def redo():
                    _, s2 = sum_sweep(m_b)
                    merge(gvec, m_b, s2)

                lax.cond(m_b < macc + 60.0, ok, redo)

            def fresh():
                @plsc.parallel_loop(0, BLK, step=L, unroll=UNROLL,
                                    carry=jnp.full((L,), NEG, jnp.float32))
                def mv(o, m):
                    return jnp.maximum(m, xbuf[pl.ds(o, L)])
                m_sc = jnp.max(mv)
                _, s = sum_sweep(m_sc)
                merge(gvec, m_sc, s)

            lax.cond(macc > -1.0e38, seeded, fresh)
            plsc.store_scatter(mbuf, [bvec], jnp.full((L,), g0, jnp.int32),
                               mask=lane0)

        def mixed():
            off = pl.multiple_of(base + b * BLK, 8)
            pltpu.sync_copy(i_hbm.at[pl.ds(off, BLK)], ibuf)

            def per_group(g, _):
                @plsc.parallel_loop(0, BLK, step=L, unroll=UNROLL,
                                    carry=jnp.full((L,), NEG, jnp.float32))
                def mv(o, m):
                    xv = xbuf[pl.ds(o, L)]
                    iv = ibuf[pl.ds(o, L)]
                    return jnp.maximum(m, jnp.where(iv == g, xv, NEG))
                m_sc = jnp.max(mv)

                @plsc.parallel_loop(0, BLK, step=L, unroll=UNROLL,
                                    carry=jnp.zeros((L,), jnp.float32))
                def sv(o, s):
                    xv = xbuf[pl.ds(o, L)]
                    iv = ibuf[pl.ds(o, L)]
                    return s + jnp.where(iv == g, jnp.exp(xv - m_sc), 0.0)
                merge(jnp.full((L,), g, jnp.int32), m_sc, jnp.sum(sv))
                return None

            lax.fori_loop(g0, g1 + 1, per_group, None)

        lax.cond(g0 == g1, uniform, mixed)

    start(0, base)

    def super_body(i, _):
        b0 = 2 * i
        start(1, base + (b0 + 1) * BLK)
        wait(0)
        compute(0, b0)
        off2 = lax.select(b0 + 2 < NBLK, base + (b0 + 2) * BLK, base)
        start(0, off2)
        wait(1)
        compute(1, b0 + 1)
        return None

    lax.fori_loop(0, HALF, super_body, None)
    wait(0)
    pltpu.sync_copy(accm, pm_hbm.at[wid])
    pltpu.sync_copy(accs, ps_hbm.at[wid])
    pltpu.sync_copy(mbuf, meta_hbm.at[wid])


_pass1 = pl.kernel(
    _p1_body,
    out_type=(
        jax.ShapeDtypeStruct((NW, G), jnp.float32),
        jax.ShapeDtypeStruct((NW, G), jnp.float32),
        jax.ShapeDtypeStruct((NW, MPAD), jnp.int32),
    ),
    mesh=_mesh,
    compiler_params=_params,
    scratch_types=[
        pltpu.VMEM((BLK,), jnp.float32),
        pltpu.VMEM((BLK,), jnp.float32),
        pltpu.VMEM((BLK,), jnp.int32),
        pltpu.VMEM((EB,), jnp.int32),
        pltpu.VMEM((EB,), jnp.int32),
        pltpu.VMEM((MPAD,), jnp.int32),
        pltpu.VMEM((G,), jnp.float32),
        pltpu.VMEM((G,), jnp.float32),
        pltpu.SemaphoreType.DMA,
        pltpu.SemaphoreType.DMA,
        pltpu.SemaphoreType.DMA,
    ],
)


def _ln(x):
    bits = plsc.bitcast(x, jnp.int32)
    e = jnp.right_shift(bits, 23) - 127
    m = plsc.bitcast((bits & 0x007FFFFF) | 0x3F800000, jnp.float32)
    big = m >= 1.5
    m = jnp.where(big, m * 0.5, m)
    e = jnp.where(big, e + 1, e)
    t = m - 1.0
    p = -1.0 / 8.0
    for k in (7, 6, 5, 4, 3, 2, 1):
        p = p * t + (1.0 / k if k % 2 == 1 else -1.0 / k)
    return e.astype(jnp.float32) * 0.6931471805599453 + t * p


def _p2_body(x_hbm, i_hbm, pm_hbm, ps_hbm, meta_hbm, o_hbm,
             xb0, xb1, ob0, ob1, ibuf, cbuf, pmb, psb, mbuf,
             sx0, sx1, so0, so1, sp):
    wid = _wid()
    base = wid * CHUNK
    xb, ob = [xb0, xb1], [ob0, ob1]
    sx, so = [sx0, sx1], [so0, so1]

    def start(j, off):
        pltpu.make_async_copy(x_hbm.at[pl.ds(off, BLK)], xb[j], sx[j]).start()

    start(0, base)
    pltpu.make_async_copy(pm_hbm, pmb, sp).start()
    pltpu.make_async_copy(ps_hbm, psb, sp).start()
    pltpu.sync_copy(meta_hbm.at[wid], mbuf)
    pltpu.make_async_copy(pm_hbm, pmb, sp).wait()
    pltpu.make_async_copy(ps_hbm, psb, sp).wait()

    for j in range(G // L):
        m = jnp.full((L,), NEG, jnp.float32)
        for w in range(NW):
            m = jnp.maximum(m, pmb[pl.ds(w * G + j * L, L)])
        s = jnp.zeros((L,), jnp.float32)
        for w in range(NW):
            s = s + psb[pl.ds(w * G + j * L, L)] * jnp.exp(
                pmb[pl.ds(w * G + j * L, L)] - m)
        cbuf[pl.ds(j * L, L)] = m + _ln(s)

    def wait(j):
        pltpu.make_async_copy(x_hbm.at[pl.ds(0, BLK)], xb[j], sx[j]).wait()

    def start_out(j, off):
        pltpu.make_async_copy(ob[j], o_hbm.at[pl.ds(off, BLK)], so[j]).start()

    def wait_out(j):
        pltpu.make_async_copy(ob[j], o_hbm.at[pl.ds(0, BLK)], so[j]).wait()

    def compute(j, b):
        xbuf, obuf = xb[j], ob[j]
        g0 = mbuf[pl.ds(b, L)][0]

        def uniform():
            cv = plsc.load_gather(cbuf, [jnp.full((L,), g0, jnp.int32)])

            @plsc.parallel_loop(0, BLK, step=L, unroll=UNROLL)
            def _(o):
                obuf[pl.ds(o, L)] = xbuf[pl.ds(o, L)] - cv

        def mixed():
            off = pl.multiple_of(base + b * BLK, 8)
            pltpu.sync_copy(i_hbm.at[pl.ds(off, BLK)], ibuf)

            @plsc.parallel_loop(0, BLK, step=L, unroll=UNROLL)
            def _(o):
                iv = ibuf[pl.ds(o, L)]
                cv = plsc.load_gather(cbuf, [iv])
                obuf[pl.ds(o, L)] = xbuf[pl.ds(o, L)] - cv

        lax.cond(g0 >= 0, uniform, mixed)

    def super_body(i, _):
        b0 = 2 * i
        start(1, base + (b0 + 1) * BLK)
        wait(0)
        lax.cond(i > 0, lambda: wait_out(0), lambda: None)
        compute(0, b0)
        start_out(0, base + b0 * BLK)
        off2 = lax.select(b0 + 2 < NBLK, base + (b0 + 2) * BLK, base)
        start(0, off2)
        wait(1)
        lax.cond(i > 0, lambda: wait_out(1), lambda: None)
        compute(1, b0 + 1)
        start_out(1, base + (b0 + 1) * BLK)
        return None

    lax.fori_loop(0, HALF, super_body, None)
    wait(0)
    wait_out(0)
    wait_out(1)


_pass2 = pl.kernel(
    _p2_body,
    out_type=jax.ShapeDtypeStruct((N,), jnp.float32),
    mesh=_mesh,
    compiler_params=_params,
    scratch_types=[
        pltpu.VMEM((BLK,), jnp.float32),
        pltpu.VMEM((BLK,), jnp.float32),
        pltpu.VMEM((BLK,), jnp.float32),
        pltpu.VMEM((BLK,), jnp.float32),
        pltpu.VMEM((BLK,), jnp.int32),
        pltpu.VMEM((G,), jnp.float32),
        pltpu.VMEM((NW * G,), jnp.float32),
        pltpu.VMEM((NW * G,), jnp.float32),
        pltpu.VMEM((MPAD,), jnp.int32),
        pltpu.SemaphoreType.DMA,
        pltpu.SemaphoreType.DMA,
        pltpu.SemaphoreType.DMA,
        pltpu.SemaphoreType.DMA,
        pltpu.SemaphoreType.DMA,
    ],
)


def kernel(logits, index):
    pm, ps, meta = _pass1(logits, index)
    return _pass2(logits, index, pm.reshape(NW * G), ps.reshape(NW * G), meta)


# speedup vs baseline: 1.0650x; 1.0650x over previous
import jax
import jax.numpy as jnp
from jax import lax
from jax.experimental import pallas as pl
from jax.experimental.pallas import tpu as pltpu
from jax.experimental.pallas import tpu_sc as plsc

N = 12_800_000
G = 128
NC, NS, L = 2, 16, 16
NW = NC * NS
CHUNK = N // NW
BLK = 10_000
NBLK = CHUNK // BLK
HALF = NBLK // 2
VPB = BLK // L
UNROLL = 16
MPAD = 80
EB = 128
NEG = -3.0e38

_mesh = plsc.VectorSubcoreMesh(core_axis_name="c", subcore_axis_name="s")
_params = pltpu.CompilerParams(needs_layout_passes=False)


def _wid():
    return lax.axis_index("s") * NC + lax.axis_index("c")


def _p1_body(x_hbm, i_hbm, pm_hbm, ps_hbm, meta_hbm,
             xb0, xb1, ibuf, ebuf, eidx, mbuf, accm, accs,
             sx0, sx1, se):
    wid = _wid()
    base = wid * CHUNK
    lane = lax.iota(jnp.int32, L)
    lane0 = lane == 0
    xb, sx = [xb0, xb1], [sx0, sx1]

    for j in range(G // L):
        accm[pl.ds(j * L, L)] = jnp.full((L,), NEG, jnp.float32)
        accs[pl.ds(j * L, L)] = jnp.zeros((L,), jnp.float32)
    for j in range(MPAD // L):
        mbuf[pl.ds(j * L, L)] = jnp.full((L,), -1, jnp.int32)

    for k in range(EB // L):
        p = k * L + lane
        b = jnp.minimum(p >> 1, NBLK - 1)
        is_end = p & 1
        eidx[pl.ds(k * L, L)] = base + b * BLK + is_end * (BLK - 1)
    pltpu.async_copy(i_hbm.at[eidx], ebuf, se).wait()

    def start(j, off):
        pltpu.make_async_copy(x_hbm.at[pl.ds(off, BLK)], xb[j], sx[j]).start()

    def wait(j):
        pltpu.make_async_copy(x_hbm.at[pl.ds(0, BLK)], xb[j], sx[j]).wait()

    def merge(gvec, m_sc, s_sc):
        mold = plsc.load_gather(accm, [gvec])
        sold = plsc.load_gather(accs, [gvec])
        mnew = jnp.maximum(mold, m_sc)
        snew = sold * jnp.exp(mold - mnew) + s_sc * jnp.exp(m_sc - mnew)
        plsc.store_scatter(accm, [gvec], mnew, mask=lane0)
        plsc.store_scatter(accs, [gvec], snew, mask=lane0)

    def compute(j, b):
        xbuf = xb[j]
        g0 = ebuf[pl.ds(2 * b, L)][0]
        g1 = ebuf[pl.ds(2 * b, L)][1]
        bvec = jnp.full((L,), b, jnp.int32)

        def uniform():
            gvec = jnp.full((L,), g0, jnp.int32)
            macc = jnp.max(plsc.load_gather(accm, [gvec]))

            def sum_sweep(shift):
                @plsc.parallel_loop(
                    0, BLK, step=L, unroll=UNROLL,
                    carry=(jnp.full((L,), NEG, jnp.float32),
                           jnp.zeros((L,), jnp.float32)))
                def ms(o, carry):
                    m, s = carry
                    xv = xbuf[pl.ds(o, L)]
                    return jnp.maximum(m, xv), s + jnp.exp(xv - shift)
                mv, sv = ms
                return jnp.max(mv), jnp.sum(sv)

            def seeded():
                m_b, s = sum_sweep(macc)

                def ok():
                    merge(gvec, macc, s)

                def redo():
                    _, s2 = sum_sweep(m_b)
                    merge(gvec, m_b, s2)

                lax.cond(m_b < macc + 60.0, ok, redo)

            def fresh():
                @plsc.parallel_loop(0, BLK, step=L, unroll=UNROLL,
                                    carry=jnp.full((L,), NEG, jnp.float32))
                def mv(o, m):
                    return jnp.maximum(m, xbuf[pl.ds(o, L)])
                m_sc = jnp.max(mv)
                _, s = sum_sweep(m_sc)
                merge(gvec, m_sc, s)

            lax.cond(macc > -1.0e38, seeded, fresh)
            plsc.store_scatter(mbuf, [bvec], jnp.full((L,), g0, jnp.int32),
                               mask=lane0)

        def mixed():
            off = pl.multiple_of(base + b * BLK, 8)
            pltpu.sync_copy(i_hbm.at[pl.ds(off, BLK)], ibuf)

            def per_group(g, _):
                @plsc.parallel_loop(0, BLK, step=L, unroll=UNROLL,
                                    carry=jnp.full((L,), NEG, jnp.float32))
                def mv(o, m):
                    xv = xbuf[pl.ds(o, L)]
                    iv = ibuf[pl.ds(o, L)]
                    return jnp.maximum(m, jnp.where(iv == g, xv, NEG))
                m_sc = jnp.max(mv)

                @plsc.parallel_loop(0, BLK, step=L, unroll=UNROLL,
                                    carry=jnp.zeros((L,), jnp.float32))
                def sv(o, s):
                    xv = xbuf[pl.ds(o, L)]
                    iv = ibuf[pl.ds(o, L)]
                    return s + jnp.where(iv == g, jnp.exp(xv - m_sc), 0.0)
                merge(jnp.full((L,), g, jnp.int32), m_sc, jnp.sum(sv))
                return None

            lax.fori_loop(g0, g1 + 1, per_group, None)

        lax.cond(g0 == g1, uniform, mixed)

    start(0, base)

    def super_body(i, _):
        b0 = 2 * i
        start(1, base + (b0 + 1) * BLK)
        wait(0)
        compute(0, b0)
        off2 = lax.select(b0 + 2 < NBLK, base + (b0 + 2) * BLK, base)
        start(0, off2)
        wait(1)
        compute(1, b0 + 1)
        return None

    lax.fori_loop(0, HALF, super_body, None)
    wait(0)
    pltpu.sync_copy(accm, pm_hbm.at[wid])
    pltpu.sync_copy(accs, ps_hbm.at[wid])
    pltpu.sync_copy(mbuf, meta_hbm.at[wid])


_pass1 = pl.kernel(
    _p1_body,
    out_type=(
        jax.ShapeDtypeStruct((NW, G), jnp.float32),
        jax.ShapeDtypeStruct((NW, G), jnp.float32),
        jax.ShapeDtypeStruct((NW, MPAD), jnp.int32),
    ),
    mesh=_mesh,
    compiler_params=_params,
    scratch_types=[
        pltpu.VMEM((BLK,), jnp.float32),
        pltpu.VMEM((BLK,), jnp.float32),
        pltpu.VMEM((BLK,), jnp.int32),
        pltpu.VMEM((EB,), jnp.int32),
        pltpu.VMEM((EB,), jnp.int32),
        pltpu.VMEM((MPAD,), jnp.int32),
        pltpu.VMEM((G,), jnp.float32),
        pltpu.VMEM((G,), jnp.float32),
        pltpu.SemaphoreType.DMA,
        pltpu.SemaphoreType.DMA,
        pltpu.SemaphoreType.DMA,
    ],
)


def _ln(x):
    bits = plsc.bitcast(x, jnp.int32)
    e = jnp.right_shift(bits, 23) - 127
    m = plsc.bitcast((bits & 0x007FFFFF) | 0x3F800000, jnp.float32)
    big = m >= 1.5
    m = jnp.where(big, m * 0.5, m)
    e = jnp.where(big, e + 1, e)
    t = m - 1.0
    p = -1.0 / 8.0
    for k in (7, 6, 5, 4, 3, 2, 1):
        p = p * t + (1.0 / k if k % 2 == 1 else -1.0 / k)
    return e.astype(jnp.float32) * 0.6931471805599453 + t * p


def _p2_body(x_hbm, i_hbm, pm_hbm, ps_hbm, meta_hbm, o_hbm,
             xb0, xb1, ob0, ob1, ibuf, cbuf, pmb, psb, mbuf,
             sx0, sx1, so0, so1, sp):
    wid = _wid()
    base = wid * CHUNK
    xb, ob = [xb0, xb1], [ob0, ob1]
    sx, so = [sx0, sx1], [so0, so1]

    def start(j, off):
        pltpu.make_async_copy(x_hbm.at[pl.ds(off, BLK)], xb[j], sx[j]).start()

    start(0, base)
    pltpu.make_async_copy(pm_hbm, pmb, sp).start()
    pltpu.make_async_copy(ps_hbm, psb, sp).start()
    pltpu.sync_copy(meta_hbm.at[wid], mbuf)
    pltpu.make_async_copy(pm_hbm, pmb, sp).wait()
    pltpu.make_async_copy(ps_hbm, psb, sp).wait()

    for j in range(G // L):
        m = jnp.full((L,), NEG, jnp.float32)
        for w in range(NW):
            m = jnp.maximum(m, pmb[pl.ds(w * G + j * L, L)])
        s = jnp.zeros((L,), jnp.float32)
        for w in range(NW):
            s = s + psb[pl.ds(w * G + j * L, L)] * jnp.exp(
                pmb[pl.ds(w * G + j * L, L)] - m)
        cbuf[pl.ds(j * L, L)] = m + _ln(s)

    def wait(j):
        pltpu.make_async_copy(x_hbm.at[pl.ds(0, BLK)], xb[j], sx[j]).wait()

    def start_out(j, off):
        pltpu.make_async_copy(ob[j], o_hbm.at[pl.ds(off, BLK)], so[j]).start()

    def wait_out(j):
        pltpu.make_async_copy(ob[j], o_hbm.at[pl.ds(0, BLK)], so[j]).wait()

    def compute(j, b):
        xbuf, obuf = xb[j], ob[j]
        g0 = mbuf[pl.ds(b, L)][0]

        def uniform():
            cv = plsc.load_gather(cbuf, [jnp.full((L,), g0, jnp.int32)])

            @plsc.parallel_loop(0, BLK, step=L, unroll=UNROLL)
            def _(o):
                obuf[pl.ds(o, L)] = xbuf[pl.ds(o, L)] - cv

        def mixed():
            off = pl.multiple_of(base + b * BLK, 8)
            pltpu.sync_copy(i_hbm.at[pl.ds(off, BLK)], ibuf)

            @plsc.parallel_loop(0, BLK, step=L, unroll=UNROLL)
            def _(o):
                iv = ibuf[pl.ds(o, L)]
                cv = plsc.load_gather(cbuf, [iv])
                obuf[pl.ds(o, L)] = xbuf[pl.ds(o, L)] - cv

        lax.cond(g0 >= 0, uniform, mixed)

    def super_body(i, _):
        b0 = 2 * i
        start(1, base + (b0 + 1) * BLK)
        wait(0)
        lax.cond(i > 0, lambda: wait_out(0), lambda: None)
        compute(0, b0)
        start_out(0, base + b0 * BLK)
        off2 = lax.select(b0 + 2 < NBLK, base + (b0 + 2) * BLK, base)
        start(0, off2)
        wait(1)
        lax.cond(i > 0, lambda: wait_out(1), lambda: None)
        compute(1, b0 + 1)
        start_out(1, base + (b0 + 1) * BLK)
        return None

    lax.fori_loop(0, HALF, super_body, None)
    wait(0)
    wait_out(0)
    wait_out(1)


_pass2 = pl.kernel(
    _p2_body,
    out_type=jax.ShapeDtypeStruct((N,), jnp.float32),
    mesh=_mesh,
    compiler_params=_params,
    scratch_types=[
        pltpu.VMEM((BLK,), jnp.float32),
        pltpu.VMEM((BLK,), jnp.float32),
        pltpu.VMEM((BLK,), jnp.float32),
        pltpu.VMEM((BLK,), jnp.float32),
        pltpu.VMEM((BLK,), jnp.int32),
        pltpu.VMEM((G,), jnp.float32),
        pltpu.VMEM((NW * G,), jnp.float32),
        pltpu.VMEM((NW * G,), jnp.float32),
        pltpu.VMEM((MPAD,), jnp.int32),
        pltpu.SemaphoreType.DMA,
        pltpu.SemaphoreType.DMA,
        pltpu.SemaphoreType.DMA,
        pltpu.SemaphoreType.DMA,
        pltpu.SemaphoreType.DMA,
    ],
)


def kernel(logits, index):
    pm, ps, meta = _pass1(logits, index)
    return _pass2(logits, index, pm.reshape(NW * G), ps.reshape(NW * G), meta)


# speedup vs baseline: 1.1385x; 1.0690x over previous
import jax
import jax.numpy as jnp
from jax import lax
from jax.experimental import pallas as pl
from jax.experimental.pallas import tpu as pltpu
from jax.experimental.pallas import tpu_sc as plsc

N = 12_800_000
G = 128
NC, NS, L = 2, 16, 16
NW = NC * NS
CHUNK = N // NW
BLK = 10_000
NBLK = CHUNK // BLK
NBUF = 4
NOUT = NBLK // NBUF
VPB = BLK // L
UNROLL = 8
MPAD = 80
EB = 128
NEG = -3.0e38

_mesh = plsc.VectorSubcoreMesh(core_axis_name="c", subcore_axis_name="s")
_params = pltpu.CompilerParams(needs_layout_passes=False)


def _wid():
    return lax.axis_index("s") * NC + lax.axis_index("c")


def _p1_body(x_hbm, i_hbm, pm_hbm, ps_hbm, meta_hbm,
             xb0, xb1, xb2, xb3, ibuf, ebuf, eidx, mbuf, accm, accs,
             sx0, sx1, sx2, sx3, se):
    wid = _wid()
    base = wid * CHUNK
    lane = lax.iota(jnp.int32, L)
    lane0 = lane == 0
    xb, sx = [xb0, xb1, xb2, xb3], [sx0, sx1, sx2, sx3]

    for j in range(G // L):
        accm[pl.ds(j * L, L)] = jnp.full((L,), NEG, jnp.float32)
        accs[pl.ds(j * L, L)] = jnp.zeros((L,), jnp.float32)
    for j in range(MPAD // L):
        mbuf[pl.ds(j * L, L)] = jnp.full((L,), -1, jnp.int32)

    for k in range(EB // L):
        p = k * L + lane
        b = jnp.minimum(p >> 1, NBLK - 1)
        is_end = p & 1
        eidx[pl.ds(k * L, L)] = base + b * BLK + is_end * (BLK - 1)
    pltpu.async_copy(i_hbm.at[eidx], ebuf, se).wait()

    def start(j, off):
        pltpu.make_async_copy(x_hbm.at[pl.ds(off, BLK)], xb[j], sx[j]).start()

    def wait(j):
        pltpu.make_async_copy(x_hbm.at[pl.ds(0, BLK)], xb[j], sx[j]).wait()

    def merge(gvec, m_sc, s_sc):
        mold = plsc.load_gather(accm, [gvec])
        sold = plsc.load_gather(accs, [gvec])
        mnew = jnp.maximum(mold, m_sc)
        snew = sold * jnp.exp(mold - mnew) + s_sc * jnp.exp(m_sc - mnew)
        plsc.store_scatter(accm, [gvec], mnew, mask=lane0)
        plsc.store_scatter(accs, [gvec], snew, mask=lane0)

    def compute(j, b):
        xbuf = xb[j]
        g0 = ebuf[pl.ds(2 * b, L)][0]
        g1 = ebuf[pl.ds(2 * b, L)][1]
        bvec = jnp.full((L,), b, jnp.int32)

        def uniform():
            gvec = jnp.full((L,), g0, jnp.int32)
            macc = jnp.max(plsc.load_gather(accm, [gvec]))

            def sum_sweep(shift):
                @plsc.parallel_loop(
                    0, BLK, step=L, unroll=UNROLL,
                    carry=(jnp.full((L,), NEG, jnp.float32),
                           jnp.zeros((L,), jnp.float32)))
                def ms(o, carry):
                    m, s = carry
                    xv = xbuf[pl.ds(o, L)]
                    return jnp.maximum(m, xv), s + jnp.exp(xv - shift)
                mv, sv = ms
                return jnp.max(mv), jnp.sum(sv)

            def seeded():
                m_b, s = sum_sweep(macc)

                def ok():
                    merge(gvec, macc, s)

                def redo():
                    _, s2 = sum_sweep(m_b)
                    merge(gvec, m_b, s2)

                lax.cond(m_b < macc + 60.0, ok, redo)

            def fresh():
                @plsc.parallel_loop(0, BLK, step=L, unroll=UNROLL,
                                    carry=jnp.full((L,), NEG, jnp.float32))
                def mv(o, m):
                    return jnp.maximum(m, xbuf[pl.ds(o, L)])
                m_sc = jnp.max(mv)
                _, s = sum_sweep(m_sc)
                merge(gvec, m_sc, s)

            lax.cond(macc > -1.0e38, seeded, fresh)
            plsc.store_scatter(mbuf, [bvec], jnp.full((L,), g0, jnp.int32),
                               mask=lane0)

        def mixed():
            off = pl.multiple_of(base + b * BLK, 8)
            pltpu.sync_copy(i_hbm.at[pl.ds(off, BLK)], ibuf)

            def per_group(g, _):
                @plsc.parallel_loop(0, BLK, step=L, unroll=UNROLL,
                                    carry=jnp.full((L,), NEG, jnp.float32))
                def mv(o, m):
                    xv = xbuf[pl.ds(o, L)]
                    iv = ibuf[pl.ds(o, L)]
                    return jnp.maximum(m, jnp.where(iv == g, xv, NEG))
                m_sc = jnp.max(mv)

                @plsc.parallel_loop(0, BLK, step=L, unroll=UNROLL,
                                    carry=jnp.zeros((L,), jnp.float32))
                def sv(o, s):
                    xv = xbuf[pl.ds(o, L)]
                    iv = ibuf[pl.ds(o, L)]
                    return s + jnp.where(iv == g, jnp.exp(xv - m_sc), 0.0)
                merge(jnp.full((L,), g, jnp.int32), m_sc, jnp.sum(sv))
                return None

            lax.fori_loop(g0, g1 + 1, per_group, None)

        lax.cond(g0 == g1, uniform, mixed)

    for j in range(NBUF - 1):
        start(j, base + j * BLK)

    def super_body(i, _):
        for j in range(NBUF):
            b = NBUF * i + j
            nxt = b + NBUF - 1
            offn = lax.select(nxt < NBLK, base + nxt * BLK, base)
            start((j + NBUF - 1) % NBUF, offn)
            wait(j)
            compute(j, b)
        return None

    lax.fori_loop(0, NOUT, super_body, None)
    for j in range(NBUF - 1):
        wait(j)
    pltpu.sync_copy(accm, pm_hbm.at[wid])
    pltpu.sync_copy(accs, ps_hbm.at[wid])
    pltpu.sync_copy(mbuf, meta_hbm.at[wid])


_pass1 = pl.kernel(
    _p1_body,
    out_type=(
        jax.ShapeDtypeStruct((NW, G), jnp.float32),
        jax.ShapeDtypeStruct((NW, G), jnp.float32),
        jax.ShapeDtypeStruct((NW, MPAD), jnp.int32),
    ),
    mesh=_mesh,
    compiler_params=_params,
    scratch_types=[
        pltpu.VMEM((BLK,), jnp.float32),
        pltpu.VMEM((BLK,), jnp.float32),
        pltpu.VMEM((BLK,), jnp.float32),
        pltpu.VMEM((BLK,), jnp.float32),
        pltpu.VMEM((BLK,), jnp.int32),
        pltpu.VMEM((EB,), jnp.int32),
        pltpu.VMEM((EB,), jnp.int32),
        pltpu.VMEM((MPAD,), jnp.int32),
        pltpu.VMEM((G,), jnp.float32),
        pltpu.VMEM((G,), jnp.float32),
        pltpu.SemaphoreType.DMA,
        pltpu.SemaphoreType.DMA,
        pltpu.SemaphoreType.DMA,
        pltpu.SemaphoreType.DMA,
        pltpu.SemaphoreType.DMA,
    ],
)


def _ln(x):
    bits = plsc.bitcast(x, jnp.int32)
    e = jnp.right_shift(bits, 23) - 127
    m = plsc.bitcast((bits & 0x007FFFFF) | 0x3F800000, jnp.float32)
    big = m >= 1.5
    m = jnp.where(big, m * 0.5, m)
    e = jnp.where(big, e + 1, e)
    t = m - 1.0
    p = -1.0 / 8.0
    for k in (7, 6, 5, 4, 3, 2, 1):
        p = p * t + (1.0 / k if k % 2 == 1 else -1.0 / k)
    return e.astype(jnp.float32) * 0.6931471805599453 + t * p


def _p2_body(x_hbm, i_hbm, pm_hbm, ps_hbm, meta_hbm, o_hbm,
             xb0, xb1, xb2, xb3, ob0, ob1, ob2, ob3,
             ibuf, cbuf, pmb, psb, mbuf,
             sx0, sx1, sx2, sx3, so0, so1, so2, so3, sp):
    wid = _wid()
    base = wid * CHUNK
    xb, ob = [xb0, xb1, xb2, xb3], [ob0, ob1, ob2, ob3]
    sx, so = [sx0, sx1, sx2, sx3], [so0, so1, so2, so3]

    def start(j, off):
        pltpu.make_async_copy(x_hbm.at[pl.ds(off, BLK)], xb[j], sx[j]).start()

    for j in range(NBUF - 1):
        start(j, base + j * BLK)
    pltpu.make_async_copy(pm_hbm, pmb, sp).start()
    pltpu.make_async_copy(ps_hbm, psb, sp).start()
    pltpu.sync_copy(meta_hbm.at[wid], mbuf)
    pltpu.make_async_copy(pm_hbm, pmb, sp).wait()
    pltpu.make_async_copy(ps_hbm, psb, sp).wait()

    for j in range(G // L):
        m = jnp.full((L,), NEG, jnp.float32)
        for w in range(NW):
            m = jnp.maximum(m, pmb[pl.ds(w * G + j * L, L)])
        s = jnp.zeros((L,), jnp.float32)
        for w in range(NW):
            s = s + psb[pl.ds(w * G + j * L, L)] * jnp.exp(
                pmb[pl.ds(w * G + j * L, L)] - m)
        cbuf[pl.ds(j * L, L)] = m + _ln(s)

    def wait(j):
        pltpu.make_async_copy(x_hbm.at[pl.ds(0, BLK)], xb[j], sx[j]).wait()

    def start_out(j, off):
        pltpu.make_async_copy(ob[j], o_hbm.at[pl.ds(off, BLK)], so[j]).start()

    def wait_out(j):
        pltpu.make_async_copy(ob[j], o_hbm.at[pl.ds(0, BLK)], so[j]).wait()

    def compute(j, b):
        xbuf, obuf = xb[j], ob[j]
        g0 = mbuf[pl.ds(b, L)][0]

        def uniform():
            cv = plsc.load_gather(cbuf, [jnp.full((L,), g0, jnp.int32)])

            @plsc.parallel_loop(0, BLK, step=L, unroll=UNROLL)
            def _(o):
                obuf[pl.ds(o, L)] = xbuf[pl.ds(o, L)] - cv

        def mixed():
            off = pl.multiple_of(base + b * BLK, 8)
            pltpu.sync_copy(i_hbm.at[pl.ds(off, BLK)], ibuf)

            @plsc.parallel_loop(0, BLK, step=L, unroll=UNROLL)
            def _(o):
                iv = ibuf[pl.ds(o, L)]
                cv = plsc.load_gather(cbuf, [iv])
                obuf[pl.ds(o, L)] = xbuf[pl.ds(o, L)] - cv

        lax.cond(g0 >= 0, uniform, mixed)

    def super_body(i, _):
        for j in range(NBUF):
            b = NBUF * i + j
            nxt = b + NBUF - 1
            offn = lax.select(nxt < NBLK, base + nxt * BLK, base)
            start((j + NBUF - 1) % NBUF, offn)
            wait(j)
            lax.cond(i > 0, lambda: wait_out(j), lambda: None)
            compute(j, b)
            start_out(j, base + b * BLK)
        return None

    lax.fori_loop(0, NOUT, super_body, None)
    for j in range(NBUF - 1):
        wait(j)
    for j in range(NBUF):
        wait_out(j)


_pass2 = pl.kernel(
    _p2_body,
    out_type=jax.ShapeDtypeStruct((N,), jnp.float32),
    mesh=_mesh,
    compiler_params=_params,
    scratch_types=(
        [pltpu.VMEM((BLK,), jnp.float32)] * 8
        + [
            pltpu.VMEM((BLK,), jnp.int32),
            pltpu.VMEM((G,), jnp.float32),
            pltpu.VMEM((NW * G,), jnp.float32),
            pltpu.VMEM((NW * G,), jnp.float32),
            pltpu.VMEM((MPAD,), jnp.int32),
        ]
        + [pltpu.SemaphoreType.DMA] * 9
    ),
)


def kernel(logits, index):
    pm, ps, meta = _pass1(logits, index)
    return _pass2(logits, index, pm.reshape(NW * G), ps.reshape(NW * G), meta)


# speedup vs baseline: 1.1401x; 1.0015x over previous
import jax
import jax.numpy as jnp
from jax import lax
from jax.experimental import pallas as pl
from jax.experimental.pallas import tpu as pltpu
from jax.experimental.pallas import tpu_sc as plsc

N = 12_800_000
G = 128
NC, NS, L = 2, 16, 16
NW = NC * NS
CHUNK = N // NW
BLK = 10_000
NBLK = CHUNK // BLK
NBUF = 4
NOUT = NBLK // NBUF
VPB = BLK // L
UNROLL = 8
MPAD = 80
EB = 128
NEG = -3.0e38

assert CHUNK % BLK == 0 and NBLK % NBUF == 0 and BLK % L == 0
assert 2 * NBLK + L <= EB <= 128 and NBLK + L <= MPAD

_mesh = plsc.VectorSubcoreMesh(core_axis_name="c", subcore_axis_name="s")
_params = pltpu.CompilerParams(needs_layout_passes=False)


def _wid():
    return lax.axis_index("s") * NC + lax.axis_index("c")


def _p1_body(x_hbm, i_hbm, pm_hbm, ps_hbm, meta_hbm,
             xb0, xb1, xb2, xb3, ibuf, ebuf, eidx, mbuf, accm, accs,
             sx0, sx1, sx2, sx3, se):
    wid = _wid()
    base = wid * CHUNK
    lane = lax.iota(jnp.int32, L)
    lane0 = lane == 0
    xb, sx = [xb0, xb1, xb2, xb3], [sx0, sx1, sx2, sx3]

    for j in range(G // L):
        accm[pl.ds(j * L, L)] = jnp.full((L,), NEG, jnp.float32)
        accs[pl.ds(j * L, L)] = jnp.zeros((L,), jnp.float32)
    for j in range(MPAD // L):
        mbuf[pl.ds(j * L, L)] = jnp.full((L,), -1, jnp.int32)

    for k in range(EB // L):
        p = k * L + lane
        b = jnp.minimum(p >> 1, NBLK - 1)
        is_end = p & 1
        eidx[pl.ds(k * L, L)] = base + b * BLK + is_end * (BLK - 1)
    pltpu.async_copy(i_hbm.at[eidx], ebuf, se).wait()

    def start(j, off):
        pltpu.make_async_copy(x_hbm.at[pl.ds(off, BLK)], xb[j], sx[j]).start()

    def wait(j):
        pltpu.make_async_copy(x_hbm.at[pl.ds(0, BLK)], xb[j], sx[j]).wait()

    def merge(gvec, m_sc, s_sc):
        mold = plsc.load_gather(accm, [gvec])
        sold = plsc.load_gather(accs, [gvec])
        mnew = jnp.maximum(mold, m_sc)
        snew = sold * jnp.exp(mold - mnew) + s_sc * jnp.exp(m_sc - mnew)
        plsc.store_scatter(accm, [gvec], mnew, mask=lane0)
        plsc.store_scatter(accs, [gvec], snew, mask=lane0)

    def compute(j, b):
        xbuf = xb[j]
        g0 = ebuf[pl.ds(2 * b, L)][0]
        g1 = ebuf[pl.ds(2 * b, L)][1]
        bvec = jnp.full((L,), b, jnp.int32)

        def uniform():
            gvec = jnp.full((L,), g0, jnp.int32)
            macc = jnp.max(plsc.load_gather(accm, [gvec]))

            def sum_sweep(shift):
                @plsc.parallel_loop(
                    0, BLK, step=L, unroll=UNROLL,
                    carry=(jnp.full((L,), NEG, jnp.float32),
                           jnp.zeros((L,), jnp.float32)))
                def ms(o, carry):
                    m, s = carry
                    xv = xbuf[pl.ds(o, L)]
                    return jnp.maximum(m, xv), s + jnp.exp(xv - shift)
                mv, sv = ms
                return jnp.max(mv), jnp.sum(sv)

            def seeded():
                m_b, s = sum_sweep(macc)

                def ok():
                    merge(gvec, macc, s)

                def redo():
                    _, s2 = sum_sweep(m_b)
                    merge(gvec, m_b, s2)

                lax.cond(m_b < macc + 60.0, ok, redo)

            def fresh():
                @plsc.parallel_loop(0, BLK, step=L, unroll=UNROLL,
                                    carry=jnp.full((L,), NEG, jnp.float32))
                def mv(o, m):
                    return jnp.maximum(m, xbuf[pl.ds(o, L)])
                m_sc = jnp.max(mv)
                _, s = sum_sweep(m_sc)
                merge(gvec, m_sc, s)

            lax.cond(macc > -1.0e38, seeded, fresh)
            plsc.store_scatter(mbuf, [bvec], jnp.full((L,), g0, jnp.int32),
                               mask=lane0)

        def mixed():
            off = pl.multiple_of(base + b * BLK, 8)
            pltpu.sync_copy(i_hbm.at[pl.ds(off, BLK)], ibuf)

            def per_group(g, _):
                @plsc.parallel_loop(0, BLK, step=L, unroll=UNROLL,
                                    carry=jnp.full((L,), NEG, jnp.float32))
                def mv(o, m):
                    xv = xbuf[pl.ds(o, L)]
                    iv = ibuf[pl.ds(o, L)]
                    return jnp.maximum(m, jnp.where(iv == g, xv, NEG))
                m_sc = jnp.max(mv)

                @plsc.parallel_loop(0, BLK, step=L, unroll=UNROLL,
                                    carry=jnp.zeros((L,), jnp.float32))
                def sv(o, s):
                    xv = xbuf[pl.ds(o, L)]
                    iv = ibuf[pl.ds(o, L)]
                    return s + jnp.where(iv == g, jnp.exp(xv - m_sc), 0.0)
                merge(jnp.full((L,), g, jnp.int32), m_sc, jnp.sum(sv))
                return None

            lax.fori_loop(g0, g1 + 1, per_group, None)

        lax.cond(g0 == g1, uniform, mixed)

    for j in range(NBUF - 1):
        start(j, base + j * BLK)

    def super_body(i, _):
        for j in range(NBUF):
            b = NBUF * i + j
            nxt = b + NBUF - 1
            offn = lax.select(nxt < NBLK, base + nxt * BLK, base)
            start((j + NBUF - 1) % NBUF, offn)
            wait(j)
            compute(j, b)
        return None

    lax.fori_loop(0, NOUT, super_body, None)
    for j in range(NBUF - 1):
        wait(j)
    pltpu.sync_copy(accm, pm_hbm.at[wid])
    pltpu.sync_copy(accs, ps_hbm.at[wid])
    pltpu.sync_copy(mbuf, meta_hbm.at[wid])


_pass1 = pl.kernel(
    _p1_body,
    out_type=(
        jax.ShapeDtypeStruct((NW, G), jnp.float32),
        jax.ShapeDtypeStruct((NW, G), jnp.float32),
        jax.ShapeDtypeStruct((NW, MPAD), jnp.int32),
    ),
    mesh=_mesh,
    compiler_params=_params,
    scratch_types=[
        pltpu.VMEM((BLK,), jnp.float32),
        pltpu.VMEM((BLK,), jnp.float32),
        pltpu.VMEM((BLK,), jnp.float32),
        pltpu.VMEM((BLK,), jnp.float32),
        pltpu.VMEM((BLK,), jnp.int32),
        pltpu.VMEM((EB,), jnp.int32),
        pltpu.VMEM((EB,), jnp.int32),
        pltpu.VMEM((MPAD,), jnp.int32),
        pltpu.VMEM((G,), jnp.float32),
        pltpu.VMEM((G,), jnp.float32),
        pltpu.SemaphoreType.DMA,
        pltpu.SemaphoreType.DMA,
        pltpu.SemaphoreType.DMA,
        pltpu.SemaphoreType.DMA,
        pltpu.SemaphoreType.DMA,
    ],
)


def _ln(x):
    bits = plsc.bitcast(x, jnp.int32)
    e = jnp.right_shift(bits, 23) - 127
    m = plsc.bitcast((bits & 0x007FFFFF) | 0x3F800000, jnp.float32)
    big = m >= 1.5
    m = jnp.where(big, m * 0.5, m)
    e = jnp.where(big, e + 1, e)
    t = m - 1.0
    p = -1.0 / 8.0
    for k in (7, 6, 5, 4, 3, 2, 1):
        p = p * t + (1.0 / k if k % 2 == 1 else -1.0 / k)
    return e.astype(jnp.float32) * 0.6931471805599453 + t * p


def _p2_body(x_hbm, i_hbm, pm_hbm, ps_hbm, meta_hbm, o_hbm,
             xb0, xb1, xb2, xb3, ob0, ob1, ob2, ob3,
             ibuf, cbuf, pmb, psb, mbuf,
             sx0, sx1, sx2, sx3, so0, so1, so2, so3, sp):
    wid = _wid()
    base = wid * CHUNK
    xb, ob = [xb0, xb1, xb2, xb3], [ob0, ob1, ob2, ob3]
    sx, so = [sx0, sx1, sx2, sx3], [so0, so1, so2, so3]

    def start(j, off):
        pltpu.make_async_copy(x_hbm.at[pl.ds(off, BLK)], xb[j], sx[j]).start()

    for j in range(NBUF - 1):
        start(j, base + j * BLK)
    pltpu.make_async_copy(pm_hbm, pmb, sp).start()
    pltpu.make_async_copy(ps_hbm, psb, sp).start()
    pltpu.sync_copy(meta_hbm.at[wid], mbuf)
    pltpu.make_async_copy(pm_hbm, pmb, sp).wait()
    pltpu.make_async_copy(ps_hbm, psb, sp).wait()

    for j in range(G // L):
        m = jnp.full((L,), NEG, jnp.float32)
        for w in range(NW):
            m = jnp.maximum(m, pmb[pl.ds(w * G + j * L, L)])
        s = jnp.zeros((L,), jnp.float32)
        for w in range(NW):
            s = s + psb[pl.ds(w * G + j * L, L)] * jnp.exp(
                pmb[pl.ds(w * G + j * L, L)] - m)
        cbuf[pl.ds(j * L, L)] = m + _ln(s)

    def wait(j):
        pltpu.make_async_copy(x_hbm.at[pl.ds(0, BLK)], xb[j], sx[j]).wait()

    def start_out(j, off):
        pltpu.make_async_copy(ob[j], o_hbm.at[pl.ds(off, BLK)], so[j]).start()

    def wait_out(j):
        pltpu.make_async_copy(ob[j], o_hbm.at[pl.ds(0, BLK)], so[j]).wait()

    def compute(j, b):
        xbuf, obuf = xb[j], ob[j]
        g0 = mbuf[pl.ds(b, L)][0]

        def uniform():
            cv = plsc.load_gather(cbuf, [jnp.full((L,), g0, jnp.int32)])

            @plsc.parallel_loop(0, BLK, step=L, unroll=UNROLL)
            def _(o):
                obuf[pl.ds(o, L)] = xbuf[pl.ds(o, L)] - cv

        def mixed():
            off = pl.multiple_of(base + b * BLK, 8)
            pltpu.sync_copy(i_hbm.at[pl.ds(off, BLK)], ibuf)

            @plsc.parallel_loop(0, BLK, step=L, unroll=UNROLL)
            def _(o):
                iv = ibuf[pl.ds(o, L)]
                cv = plsc.load_gather(cbuf, [iv])
                obuf[pl.ds(o, L)] = xbuf[pl.ds(o, L)] - cv

        lax.cond(g0 >= 0, uniform, mixed)

    def super_body(i, _):
        for j in range(NBUF):
            b = NBUF * i + j
            nxt = b + NBUF - 1
            offn = lax.select(nxt < NBLK, base + nxt * BLK, base)
            start((j + NBUF - 1) % NBUF, offn)
            wait(j)
            lax.cond(i > 0, lambda: wait_out(j), lambda: None)
            compute(j, b)
            start_out(j, base + b * BLK)
        return None

    lax.fori_loop(0, NOUT, super_body, None)
    for j in range(NBUF - 1):
        wait(j)
    for j in range(NBUF):
        wait_out(j)


_pass2 = pl.kernel(
    _p2_body,
    out_type=jax.ShapeDtypeStruct((N,), jnp.float32),
    mesh=_mesh,
    compiler_params=_params,
    scratch_types=(
        [pltpu.VMEM((BLK,), jnp.float32)] * 8
        + [
            pltpu.VMEM((BLK,), jnp.int32),
            pltpu.VMEM((G,), jnp.float32),
            pltpu.VMEM((NW * G,), jnp.float32),
            pltpu.VMEM((NW * G,), jnp.float32),
            pltpu.VMEM((MPAD,), jnp.int32),
        ]
        + [pltpu.SemaphoreType.DMA] * 9
    ),
)


def kernel(logits, index):
    pm, ps, meta = _pass1(logits, index)
    return _pass2(logits, index, pm.reshape(NW * G), ps.reshape(NW * G), meta)
